# Initial kernel scaffold; baseline (speedup 1.0000x reference)
import jax
import jax.numpy as jnp
from jax.experimental import pallas as pl


def kernel(x, W_gate, b_gate, H, W):
    raise NotImplementedError("write your pallas kernel here")


# speedup vs baseline: 3.4062x; 3.4062x over previous
import math

import jax
import jax.numpy as jnp
from jax.experimental import pallas as pl


def _body(x_ref, wt_ref, b_ref, o_ref):
    N, C = x_ref.shape[1], x_ref.shape[2]
    Hs = int(math.isqrt(N))
    Hr = Hs // 4
    n2 = (Hs // 2) * (Hs // 2)
    n1 = Hr * Hr

    xs = x_ref[0]
    u = xs.reshape(N // 2, 2, C).sum(axis=1)
    p2 = (u.reshape(Hs // 2, 2, Hs // 2, C).sum(axis=1) * 0.25).reshape(n2, C)
    u2 = p2.reshape(n2 // 2, 2, C).sum(axis=1)
    p1 = (u2.reshape(Hr, 2, Hr, C).sum(axis=1) * 0.25).reshape(n1, C)

    logits = jax.lax.dot_general(
        p1, wt_ref[...], (((1,), (0,)), ((), ())),
        preferred_element_type=jnp.float32) + b_ref[...]
    l0, l1, l2 = logits[:, 0:1], logits[:, 1:2], logits[:, 2:3]
    one = jnp.float32(1.0)
    zero = jnp.float32(0.0)
    m0 = jnp.where((l0 >= l1) & (l0 >= l2), one, zero)
    m1 = jnp.where((l1 > l0) & (l1 >= l2), one, zero)
    m2 = jnp.where((l2 > l0) & (l2 > l1), one, zero)

    o_ref[0, 0:n1] = p1 * m0
    o2 = p2.reshape(Hr, 2, Hr, 2, C) * m1.reshape(Hr, 1, Hr, 1, 1)
    o_ref[0, n1:n1 + n2] = o2.reshape(n2, C)
    o3 = xs.reshape(Hr, 4, Hr, 4, C) * m2.reshape(Hr, 1, Hr, 1, 1)
    o_ref[0, n1 + n2:n1 + n2 + N] = o3.reshape(N, C)


def kernel(x, W_gate, b_gate, H, W):
    del H, W
    B, N, C = x.shape
    Hs = int(math.isqrt(N))
    Hr = Hs // 4
    n_out = Hr * Hr + (Hs // 2) * (Hs // 2) + N
    wt = W_gate.T
    b2 = b_gate.reshape(1, -1)
    return pl.pallas_call(
        _body,
        grid=(B,),
        in_specs=[
            pl.BlockSpec((1, N, C), lambda b: (b, 0, 0)),
            pl.BlockSpec((C, W_gate.shape[0]), lambda b: (0, 0)),
            pl.BlockSpec((1, W_gate.shape[0]), lambda b: (0, 0)),
        ],
        out_specs=pl.BlockSpec((1, n_out, C), lambda b: (b, 0, 0)),
        out_shape=jax.ShapeDtypeStruct((B, n_out, C), x.dtype),
    )(x, wt, b2)
